# Initial kernel scaffold; baseline (speedup 1.0000x reference)
import functools

import jax
import jax.numpy as jnp
from jax import lax
from jax.experimental import pallas as pl
from jax.experimental.pallas import tpu as pltpu
from jax.experimental.pallas import tpu_sc as plsc

N_P = 50000
N_A = 50000
E = 800000
D = 64

NC = 2
NS = 16

EPAD = 802816
EROWS = EPAD // 128
UPT = EROWS // NS
K = 8
CH = UPT // K
NPAD = 50176
RPT = NPAD // NS

_MESH = plsc.VectorSubcoreMesh(core_axis_name="c", subcore_axis_name="s",
                               num_cores=NC, num_subcores=NS)


@functools.partial(
    pl.kernel,
    out_type=jax.ShapeDtypeStruct((NC, NPAD, 32), jnp.float32),
    mesh=_MESH,
    scratch_types=[
        pltpu.VMEM((K, 128), jnp.int32),
        pltpu.VMEM((K, 128), jnp.int32),
        pltpu.VMEM((K, 128, 32), jnp.float32),
        pltpu.VMEM_SHARED((NPAD, 32), jnp.float32),
        pltpu.SemaphoreType.DMA,
        pltpu.SemaphoreType.DMA,
    ],
)
def _seg_sum_sc(tbl_lo, tbl_hi, src_hbm, dst_hbm, zeros_hbm, out_hbm,
                src_v, dst_v, rows_v, acc, sem_g, sem_s):
    c = lax.axis_index("c")
    s = lax.axis_index("s")

    pltpu.sync_copy(zeros_hbm.at[pl.ds(s * RPT, RPT)], acc.at[pl.ds(s * RPT, RPT)])
    plsc.subcore_barrier()

    def run(tbl):
        @pl.loop(0, CH)
        def _(ci):
            base = s * UPT + ci * K
            pltpu.sync_copy(src_hbm.at[pl.ds(base, K)], src_v)
            pltpu.sync_copy(dst_hbm.at[pl.ds(base, K)], dst_v)
            gathers = [
                pltpu.async_copy(tbl.at[src_v.at[j]], rows_v.at[j], sem_g)
                for j in range(K)
            ]
            for g in gathers:
                g.wait()
            adds = [
                pltpu.async_copy(rows_v.at[j], acc.at[dst_v.at[j]], sem_s,
                                 add=True)
                for j in range(K)
            ]
            for a in adds:
                a.wait()

    @pl.when(c == 0)
    def _():
        run(tbl_lo)

    @pl.when(c == 1)
    def _():
        run(tbl_hi)

    plsc.subcore_barrier()
    pltpu.sync_copy(acc.at[pl.ds(s * RPT, RPT)],
                    out_hbm.at[c, pl.ds(s * RPT, RPT)])


@functools.partial(
    pl.kernel,
    out_type=[jax.ShapeDtypeStruct((NPAD, 16), jnp.float32),
              jax.ShapeDtypeStruct((NPAD, 16), jnp.float32)],
    mesh=_MESH,
    scratch_types=[
        pltpu.VMEM((K, 128), jnp.int32),
        pltpu.VMEM((128, 16), jnp.float32),
        pltpu.VMEM_SHARED((NPAD, 16), jnp.float32),
        pltpu.SemaphoreType.DMA,
    ],
)
def _degrees_sc(dstA_hbm, dstB_hbm, ones_hbm, zeros_hbm, outA_hbm, outB_hbm,
                dst_v, ones_v, acc, sem):
    c = lax.axis_index("c")
    s = lax.axis_index("s")

    pltpu.sync_copy(zeros_hbm.at[pl.ds(s * RPT, RPT)], acc.at[pl.ds(s * RPT, RPT)])
    pltpu.sync_copy(ones_hbm, ones_v)
    plsc.subcore_barrier()

    def run(dst_hbm):
        @pl.loop(0, CH)
        def _(ci):
            base = s * UPT + ci * K
            pltpu.sync_copy(dst_hbm.at[pl.ds(base, K)], dst_v)
            adds = [
                pltpu.async_copy(ones_v, acc.at[dst_v.at[j]], sem, add=True)
                for j in range(K)
            ]
            for a in adds:
                a.wait()

    @pl.when(c == 0)
    def _():
        run(dstA_hbm)

    @pl.when(c == 1)
    def _():
        run(dstB_hbm)

    plsc.subcore_barrier()

    @pl.when(c == 0)
    def _():
        pltpu.sync_copy(acc.at[pl.ds(s * RPT, RPT)], outA_hbm.at[pl.ds(s * RPT, RPT)])

    @pl.when(c == 1)
    def _():
        pltpu.sync_copy(acc.at[pl.ds(s * RPT, RPT)], outB_hbm.at[pl.ds(s * RPT, RPT)])


_RB = 1000


def _fuse_body(x1_ref, x2_ref, x3_ref, w1_ref, w2_ref, w3_ref, a_ref,
               hP_ref, hPs_ref, hA_ref, hAs_ref):
    m1 = jnp.dot(x1_ref[...], w1_ref[...], preferred_element_type=jnp.float32)
    m2 = jnp.dot(x2_ref[...], w2_ref[...], preferred_element_type=jnp.float32)
    a = a_ref[...]
    s1 = jnp.sum(a * jnp.tanh(m1), axis=-1)
    s2 = jnp.sum(a * jnp.tanh(m2), axis=-1)
    l1 = jnp.where(s1 >= 0, s1, 0.2 * s1)
    l2 = jnp.where(s2 >= 0, s2, 0.2 * s2)
    w = jax.nn.sigmoid(l1 - l2)[:, None]
    h = w * m1 + (1.0 - w) * m2
    hP_ref[...] = h
    hPs_ref[0] = h[:, :32]
    hPs_ref[1] = h[:, 32:]
    mA = jnp.dot(x3_ref[...], w3_ref[...], preferred_element_type=jnp.float32)
    hA_ref[...] = mA
    hAs_ref[0] = mA[:, :32]
    hAs_ref[1] = mA[:, 32:]


def _fuse_tc(x1, x2, x3, w1, w2, w3, a):
    grid = (N_P // _RB,)
    return pl.pallas_call(
        _fuse_body,
        grid=grid,
        in_specs=[
            pl.BlockSpec((_RB, 256), lambda i: (i, 0)),
            pl.BlockSpec((_RB, 384), lambda i: (i, 0)),
            pl.BlockSpec((_RB, 256), lambda i: (i, 0)),
            pl.BlockSpec((256, D), lambda i: (0, 0)),
            pl.BlockSpec((384, D), lambda i: (0, 0)),
            pl.BlockSpec((256, D), lambda i: (0, 0)),
            pl.BlockSpec((1, D), lambda i: (0, 0)),
        ],
        out_specs=[
            pl.BlockSpec((_RB, D), lambda i: (i, 0)),
            pl.BlockSpec((NC, _RB, 32), lambda i: (0, i, 0)),
            pl.BlockSpec((_RB, D), lambda i: (i, 0)),
            pl.BlockSpec((NC, _RB, 32), lambda i: (0, i, 0)),
        ],
        out_shape=[
            jax.ShapeDtypeStruct((N_P, D), jnp.float32),
            jax.ShapeDtypeStruct((NC, N_P, 32), jnp.float32),
            jax.ShapeDtypeStruct((N_A, D), jnp.float32),
            jax.ShapeDtypeStruct((NC, N_A, 32), jnp.float32),
        ],
    )(x1, x2, x3, w1, w2, w3, a)


_NB = 1024


def _norm_split_body(s_ref, deg_ref, o_ref):
    d = jnp.maximum(deg_ref[...][:, :1], 1.0)
    o_ref[0] = s_ref[0] / d
    o_ref[1] = s_ref[1] / d


def _norm_split_tc(sums, deg):
    return pl.pallas_call(
        _norm_split_body,
        grid=(NPAD // _NB,),
        in_specs=[
            pl.BlockSpec((NC, _NB, 32), lambda i: (0, i, 0)),
            pl.BlockSpec((_NB, 16), lambda i: (i, 0)),
        ],
        out_specs=pl.BlockSpec((NC, _NB, 32), lambda i: (0, i, 0)),
        out_shape=jax.ShapeDtypeStruct((NC, NPAD, 32), jnp.float32),
    )(sums, deg)


def _norm_final_body(sa_ref, sb_ref, deg_ref, oa_ref, ob_ref):
    d = jnp.maximum(deg_ref[...][:, :1], 1.0)
    oa_ref[...] = jnp.concatenate([sa_ref[0] / d, sa_ref[1] / d], axis=1)
    ob_ref[...] = jnp.concatenate([sb_ref[0] / d, sb_ref[1] / d], axis=1)


def _norm_final_tc(sums_a, sums_b, deg):
    return pl.pallas_call(
        _norm_final_body,
        grid=(NPAD // _NB,),
        in_specs=[
            pl.BlockSpec((NC, _NB, 32), lambda i: (0, i, 0)),
            pl.BlockSpec((NC, _NB, 32), lambda i: (0, i, 0)),
            pl.BlockSpec((_NB, 16), lambda i: (i, 0)),
        ],
        out_specs=[
            pl.BlockSpec((_NB, D), lambda i: (i, 0)),
            pl.BlockSpec((_NB, D), lambda i: (i, 0)),
        ],
        out_shape=[
            jax.ShapeDtypeStruct((NPAD, D), jnp.float32),
            jax.ShapeDtypeStruct((NPAD, D), jnp.float32),
        ],
    )(sums_a, sums_b, deg)


def _pad_idx(idx, fill):
    idx = idx.astype(jnp.int32)
    pad = jnp.full((EPAD - E,), fill, jnp.int32)
    return jnp.concatenate([idx, pad]).reshape(EROWS, 128)


def kernel(feat_P_PA, feat_P_PAP, feat_A_AP, src_AP, dst_AP, src_PA, dst_PA,
           W_PA, W_PAP, W_AP, a_P, a_A):
    srcAP = _pad_idx(src_AP, 0)
    dstAP = _pad_idx(dst_AP, N_P)
    srcPA = _pad_idx(src_PA, 0)
    dstPA = _pad_idx(dst_PA, N_A)

    zeros32 = jnp.zeros((NPAD, 32), jnp.float32)
    zeros16 = jnp.zeros((NPAD, 16), jnp.float32)
    ones16 = jnp.ones((128, 16), jnp.float32)

    h_P, h_P_split, h_A, h_A_split = _fuse_tc(
        feat_P_PA, feat_P_PAP, feat_A_AP, W_PA, W_PAP, W_AP, a_P)

    deg_P, deg_A = _degrees_sc(dstAP, dstPA, ones16, zeros16)

    s_PA = _seg_sum_sc(h_A_split[0], h_A_split[1], srcAP, dstAP, zeros32)
    s_AP = _seg_sum_sc(h_P_split[0], h_P_split[1], srcPA, dstPA, zeros32)
    f_AP_split = _norm_split_tc(s_AP, deg_A)

    s_PAP = _seg_sum_sc(f_AP_split[0], f_AP_split[1], srcAP, dstAP, zeros32)

    f_PA, f_PAP = _norm_final_tc(s_PA, s_PAP, deg_P)

    return jnp.stack([h_P, f_PA[:N_P], f_PAP[:N_P]])


# speedup vs baseline: 5.7094x; 5.7094x over previous
import functools

import jax
import jax.numpy as jnp
from jax import lax
from jax.experimental import pallas as pl
from jax.experimental.pallas import tpu as pltpu
from jax.experimental.pallas import tpu_sc as plsc

N_P = 50000
N_A = 50000
E = 800000
D = 64

NC = 2
NS = 16

EPAD = 802816
EROWS = EPAD // 128
UPT = EROWS // NS
K = 4
CH = UPT // K
NPAD = 50176
RPT = NPAD // NS

_MESH = plsc.VectorSubcoreMesh(core_axis_name="c", subcore_axis_name="s",
                               num_cores=NC, num_subcores=NS)
_SC_PARAMS = pltpu.CompilerParams(use_tc_tiling_on_sc=False)


@functools.partial(
    pl.kernel,
    out_type=jax.ShapeDtypeStruct((NC, NPAD, 32), jnp.float32),
    mesh=_MESH,
    scratch_types=[
        pltpu.VMEM((K, 128), jnp.int32),
        pltpu.VMEM((K, 128), jnp.int32),
        pltpu.VMEM((K, 128, 32), jnp.float32),
        pltpu.VMEM_SHARED((NPAD, 32), jnp.float32),
        pltpu.SemaphoreType.DMA,
        pltpu.SemaphoreType.DMA,
    ],
    compiler_params=_SC_PARAMS,
)
def _seg_sum_sc(tbl_lo, tbl_hi, src_hbm, dst_hbm, zeros_hbm, out_hbm,
                src_v, dst_v, rows_v, acc, sem_g, sem_s):
    c = lax.axis_index("c")
    s = lax.axis_index("s")

    pltpu.sync_copy(zeros_hbm.at[pl.ds(s * RPT, RPT)], acc.at[pl.ds(s * RPT, RPT)])
    plsc.subcore_barrier()

    def run(tbl):
        @pl.loop(0, CH)
        def _(ci):
            base = s * UPT + ci * K
            pltpu.sync_copy(src_hbm.at[pl.ds(base, K)], src_v)
            pltpu.sync_copy(dst_hbm.at[pl.ds(base, K)], dst_v)
            gathers = [
                pltpu.async_copy(tbl.at[src_v.at[j]], rows_v.at[j], sem_g)
                for j in range(K)
            ]
            for g in gathers:
                g.wait()
            adds = [
                pltpu.async_copy(rows_v.at[j], acc.at[dst_v.at[j]], sem_s,
                                 add=True)
                for j in range(K)
            ]
            for a in adds:
                a.wait()

    @pl.when(c == 0)
    def _():
        run(tbl_lo)

    @pl.when(c == 1)
    def _():
        run(tbl_hi)

    plsc.subcore_barrier()
    pltpu.sync_copy(acc.at[pl.ds(s * RPT, RPT)],
                    out_hbm.at[c, pl.ds(s * RPT, RPT)])


@functools.partial(
    pl.kernel,
    out_type=[jax.ShapeDtypeStruct((NPAD, 16), jnp.float32),
              jax.ShapeDtypeStruct((NPAD, 16), jnp.float32)],
    mesh=_MESH,
    scratch_types=[
        pltpu.VMEM((K, 128), jnp.int32),
        pltpu.VMEM((128, 16), jnp.float32),
        pltpu.VMEM_SHARED((NPAD, 16), jnp.float32),
        pltpu.SemaphoreType.DMA,
    ],
    compiler_params=_SC_PARAMS,
)
def _degrees_sc(dstA_hbm, dstB_hbm, ones_hbm, zeros_hbm, outA_hbm, outB_hbm,
                dst_v, ones_v, acc, sem):
    c = lax.axis_index("c")
    s = lax.axis_index("s")

    pltpu.sync_copy(zeros_hbm.at[pl.ds(s * RPT, RPT)], acc.at[pl.ds(s * RPT, RPT)])
    pltpu.sync_copy(ones_hbm, ones_v)
    plsc.subcore_barrier()

    def run(dst_hbm):
        @pl.loop(0, CH)
        def _(ci):
            base = s * UPT + ci * K
            pltpu.sync_copy(dst_hbm.at[pl.ds(base, K)], dst_v)
            adds = [
                pltpu.async_copy(ones_v, acc.at[dst_v.at[j]], sem, add=True)
                for j in range(K)
            ]
            for a in adds:
                a.wait()

    @pl.when(c == 0)
    def _():
        run(dstA_hbm)

    @pl.when(c == 1)
    def _():
        run(dstB_hbm)

    plsc.subcore_barrier()

    @pl.when(c == 0)
    def _():
        pltpu.sync_copy(acc.at[pl.ds(s * RPT, RPT)], outA_hbm.at[pl.ds(s * RPT, RPT)])

    @pl.when(c == 1)
    def _():
        pltpu.sync_copy(acc.at[pl.ds(s * RPT, RPT)], outB_hbm.at[pl.ds(s * RPT, RPT)])


_RB = 1000


def _fuse_body(x1_ref, x2_ref, x3_ref, w1_ref, w2_ref, w3_ref, a_ref,
               hP_ref, hPs_ref, hA_ref, hAs_ref):
    m1 = jnp.dot(x1_ref[...], w1_ref[...], preferred_element_type=jnp.float32)
    m2 = jnp.dot(x2_ref[...], w2_ref[...], preferred_element_type=jnp.float32)
    a = a_ref[...]
    s1 = jnp.sum(a * jnp.tanh(m1), axis=-1)
    s2 = jnp.sum(a * jnp.tanh(m2), axis=-1)
    l1 = jnp.where(s1 >= 0, s1, 0.2 * s1)
    l2 = jnp.where(s2 >= 0, s2, 0.2 * s2)
    w = jax.nn.sigmoid(l1 - l2)[:, None]
    h = w * m1 + (1.0 - w) * m2
    hP_ref[...] = h
    hPs_ref[0] = h[:, :32]
    hPs_ref[1] = h[:, 32:]
    mA = jnp.dot(x3_ref[...], w3_ref[...], preferred_element_type=jnp.float32)
    hA_ref[...] = mA
    hAs_ref[0] = mA[:, :32]
    hAs_ref[1] = mA[:, 32:]


def _fuse_tc(x1, x2, x3, w1, w2, w3, a):
    grid = (N_P // _RB,)
    return pl.pallas_call(
        _fuse_body,
        grid=grid,
        in_specs=[
            pl.BlockSpec((_RB, 256), lambda i: (i, 0)),
            pl.BlockSpec((_RB, 384), lambda i: (i, 0)),
            pl.BlockSpec((_RB, 256), lambda i: (i, 0)),
            pl.BlockSpec((256, D), lambda i: (0, 0)),
            pl.BlockSpec((384, D), lambda i: (0, 0)),
            pl.BlockSpec((256, D), lambda i: (0, 0)),
            pl.BlockSpec((1, D), lambda i: (0, 0)),
        ],
        out_specs=[
            pl.BlockSpec((_RB, D), lambda i: (i, 0)),
            pl.BlockSpec((NC, _RB, 32), lambda i: (0, i, 0)),
            pl.BlockSpec((_RB, D), lambda i: (i, 0)),
            pl.BlockSpec((NC, _RB, 32), lambda i: (0, i, 0)),
        ],
        out_shape=[
            jax.ShapeDtypeStruct((N_P, D), jnp.float32),
            jax.ShapeDtypeStruct((NC, N_P, 32), jnp.float32),
            jax.ShapeDtypeStruct((N_A, D), jnp.float32),
            jax.ShapeDtypeStruct((NC, N_A, 32), jnp.float32),
        ],
    )(x1, x2, x3, w1, w2, w3, a)


_NB = 1024


def _norm_split_body(s_ref, deg_ref, o_ref):
    d = jnp.maximum(deg_ref[...][:, :1], 1.0)
    o_ref[0] = s_ref[0] / d
    o_ref[1] = s_ref[1] / d


def _norm_split_tc(sums, deg):
    return pl.pallas_call(
        _norm_split_body,
        grid=(NPAD // _NB,),
        in_specs=[
            pl.BlockSpec((NC, _NB, 32), lambda i: (0, i, 0)),
            pl.BlockSpec((_NB, 16), lambda i: (i, 0)),
        ],
        out_specs=pl.BlockSpec((NC, _NB, 32), lambda i: (0, i, 0)),
        out_shape=jax.ShapeDtypeStruct((NC, NPAD, 32), jnp.float32),
    )(sums, deg)


def _norm_final_body(sa_ref, sb_ref, deg_ref, oa_ref, ob_ref):
    d = jnp.maximum(deg_ref[...][:, :1], 1.0)
    oa_ref[...] = jnp.concatenate([sa_ref[0] / d, sa_ref[1] / d], axis=1)
    ob_ref[...] = jnp.concatenate([sb_ref[0] / d, sb_ref[1] / d], axis=1)


def _norm_final_tc(sums_a, sums_b, deg):
    return pl.pallas_call(
        _norm_final_body,
        grid=(NPAD // _NB,),
        in_specs=[
            pl.BlockSpec((NC, _NB, 32), lambda i: (0, i, 0)),
            pl.BlockSpec((NC, _NB, 32), lambda i: (0, i, 0)),
            pl.BlockSpec((_NB, 16), lambda i: (i, 0)),
        ],
        out_specs=[
            pl.BlockSpec((_NB, D), lambda i: (i, 0)),
            pl.BlockSpec((_NB, D), lambda i: (i, 0)),
        ],
        out_shape=[
            jax.ShapeDtypeStruct((NPAD, D), jnp.float32),
            jax.ShapeDtypeStruct((NPAD, D), jnp.float32),
        ],
    )(sums_a, sums_b, deg)


def _pad_idx(idx, fill):
    idx = idx.astype(jnp.int32)
    pad = jnp.full((EPAD - E,), fill, jnp.int32)
    return jnp.concatenate([idx, pad]).reshape(EROWS, 128)


def kernel(feat_P_PA, feat_P_PAP, feat_A_AP, src_AP, dst_AP, src_PA, dst_PA,
           W_PA, W_PAP, W_AP, a_P, a_A):
    srcAP = _pad_idx(src_AP, 0)
    dstAP = _pad_idx(dst_AP, N_P)
    srcPA = _pad_idx(src_PA, 0)
    dstPA = _pad_idx(dst_PA, N_A)

    zeros32 = jnp.zeros((NPAD, 32), jnp.float32)
    zeros16 = jnp.zeros((NPAD, 16), jnp.float32)
    ones16 = jnp.ones((128, 16), jnp.float32)

    h_P, h_P_split, h_A, h_A_split = _fuse_tc(
        feat_P_PA, feat_P_PAP, feat_A_AP, W_PA, W_PAP, W_AP, a_P)

    deg_P, deg_A = _degrees_sc(dstAP, dstPA, ones16, zeros16)

    s_PA = _seg_sum_sc(h_A_split[0], h_A_split[1], srcAP, dstAP, zeros32)
    s_AP = _seg_sum_sc(h_P_split[0], h_P_split[1], srcPA, dstPA, zeros32)
    f_AP_split = _norm_split_tc(s_AP, deg_A)

    s_PAP = _seg_sum_sc(f_AP_split[0], f_AP_split[1], srcAP, dstAP, zeros32)

    f_PA, f_PAP = _norm_final_tc(s_PA, s_PAP, deg_P)

    return jnp.stack([h_P, f_PA[:N_P], f_PAP[:N_P]])


# speedup vs baseline: 6.8315x; 1.1965x over previous
import functools

import jax
import jax.numpy as jnp
from jax import lax
from jax.experimental import pallas as pl
from jax.experimental.pallas import tpu as pltpu
from jax.experimental.pallas import tpu_sc as plsc

N_P = 50000
N_A = 50000
E = 800000
D = 64

NC = 2
NS = 16

EPAD = 802816
EROWS = EPAD // 128
UPT = EROWS // NS
K = 2
CH = UPT // K
NPAD = 50176
RPT = NPAD // NS

_MESH = plsc.VectorSubcoreMesh(core_axis_name="c", subcore_axis_name="s",
                               num_cores=NC, num_subcores=NS)
_SC_PARAMS = pltpu.CompilerParams(use_tc_tiling_on_sc=False)


@functools.partial(
    pl.kernel,
    out_type=jax.ShapeDtypeStruct((NC, NPAD, 32), jnp.float32),
    mesh=_MESH,
    scratch_types=[
        pltpu.VMEM((K, 128), jnp.int32),
        pltpu.VMEM((K, 128), jnp.int32),
        pltpu.VMEM((K, 128), jnp.int32),
        pltpu.VMEM((K, 128), jnp.int32),
        pltpu.VMEM((K, 128, 32), jnp.float32),
        pltpu.VMEM((K, 128, 32), jnp.float32),
        pltpu.VMEM_SHARED((NPAD, 32), jnp.float32),
        pltpu.SemaphoreType.DMA,
        pltpu.SemaphoreType.DMA,
        pltpu.SemaphoreType.DMA,
    ],
    compiler_params=_SC_PARAMS,
)
def _seg_sum_sc(tbl_lo, tbl_hi, src_hbm, dst_hbm, zeros_hbm, out_hbm,
                srcA, dstA, srcB, dstB, rowsA, rowsB, acc,
                sem_i, sem_g, sem_s):
    c = lax.axis_index("c")
    s = lax.axis_index("s")

    pltpu.sync_copy(zeros_hbm.at[pl.ds(s * RPT, RPT)], acc.at[pl.ds(s * RPT, RPT)])
    plsc.subcore_barrier()

    def run(tbl):
        base = s * UPT

        def fire_idx(ci, sv, dv):
            pltpu.async_copy(src_hbm.at[pl.ds(base + ci * K, K)], sv, sem_i)
            pltpu.async_copy(dst_hbm.at[pl.ds(base + ci * K, K)], dv, sem_i)

        def drain_idx(sv, dv):
            pltpu.make_async_copy(src_hbm.at[pl.ds(base, K)], sv, sem_i).wait()
            pltpu.make_async_copy(dst_hbm.at[pl.ds(base, K)], dv, sem_i).wait()

        def fire_gathers(sv, rows):
            for j in range(K):
                pltpu.async_copy(tbl.at[sv.at[j]], rows.at[j], sem_g)

        def drain_gathers(sv, rows):
            for j in range(K):
                pltpu.make_async_copy(tbl.at[sv.at[j]], rows.at[j], sem_g).wait()

        def fire_adds(dv, rows):
            for j in range(K):
                pltpu.async_copy(rows.at[j], acc.at[dv.at[j]], sem_s, add=True)

        def drain_adds(dv, rows):
            for j in range(K):
                pltpu.make_async_copy(rows.at[j], acc.at[dv.at[j]], sem_s).wait()

        def step(ci, sv, dv, rows, nsv, ndv, nrows):
            drain_idx(nsv, ndv)
            drain_gathers(sv, rows)
            fire_adds(dv, rows)
            fire_gathers(nsv, nrows)
            drain_adds(dv, rows)
            fire_idx(ci + 2, sv, dv)

        pltpu.sync_copy(src_hbm.at[pl.ds(base, K)], srcA)
        pltpu.sync_copy(dst_hbm.at[pl.ds(base, K)], dstA)
        fire_gathers(srcA, rowsA)
        fire_idx(1, srcB, dstB)

        @pl.loop(0, CH - 2, step=2)
        def _(ci):
            step(ci, srcA, dstA, rowsA, srcB, dstB, rowsB)
            step(ci + 1, srcB, dstB, rowsB, srcA, dstA, rowsA)

        drain_idx(srcB, dstB)
        drain_gathers(srcA, rowsA)
        fire_adds(dstA, rowsA)
        fire_gathers(srcB, rowsB)
        drain_adds(dstA, rowsA)
        drain_gathers(srcB, rowsB)
        fire_adds(dstB, rowsB)
        drain_adds(dstB, rowsB)

    @pl.when(c == 0)
    def _():
        run(tbl_lo)

    @pl.when(c == 1)
    def _():
        run(tbl_hi)

    plsc.subcore_barrier()
    pltpu.sync_copy(acc.at[pl.ds(s * RPT, RPT)],
                    out_hbm.at[c, pl.ds(s * RPT, RPT)])


_KD = 4
_CHD = UPT // _KD


@functools.partial(
    pl.kernel,
    out_type=[jax.ShapeDtypeStruct((NPAD, 16), jnp.float32),
              jax.ShapeDtypeStruct((NPAD, 16), jnp.float32)],
    mesh=_MESH,
    scratch_types=[
        pltpu.VMEM((_KD, 128), jnp.int32),
        pltpu.VMEM((128, 16), jnp.float32),
        pltpu.VMEM_SHARED((NPAD, 16), jnp.float32),
        pltpu.SemaphoreType.DMA,
    ],
    compiler_params=_SC_PARAMS,
)
def _degrees_sc(dstA_hbm, dstB_hbm, ones_hbm, zeros_hbm, outA_hbm, outB_hbm,
                dst_v, ones_v, acc, sem):
    c = lax.axis_index("c")
    s = lax.axis_index("s")

    pltpu.sync_copy(zeros_hbm.at[pl.ds(s * RPT, RPT)], acc.at[pl.ds(s * RPT, RPT)])
    pltpu.sync_copy(ones_hbm, ones_v)
    plsc.subcore_barrier()

    def run(dst_hbm):
        @pl.loop(0, _CHD)
        def _(ci):
            base = s * UPT + ci * _KD
            pltpu.sync_copy(dst_hbm.at[pl.ds(base, _KD)], dst_v)
            adds = [
                pltpu.async_copy(ones_v, acc.at[dst_v.at[j]], sem, add=True)
                for j in range(_KD)
            ]
            for a in adds:
                a.wait()

    @pl.when(c == 0)
    def _():
        run(dstA_hbm)

    @pl.when(c == 1)
    def _():
        run(dstB_hbm)

    plsc.subcore_barrier()

    @pl.when(c == 0)
    def _():
        pltpu.sync_copy(acc.at[pl.ds(s * RPT, RPT)], outA_hbm.at[pl.ds(s * RPT, RPT)])

    @pl.when(c == 1)
    def _():
        pltpu.sync_copy(acc.at[pl.ds(s * RPT, RPT)], outB_hbm.at[pl.ds(s * RPT, RPT)])


_RB = 1000


def _fuse_body(x1_ref, x2_ref, x3_ref, w1_ref, w2_ref, w3_ref, a_ref,
               hP_ref, hPs_ref, hA_ref, hAs_ref):
    m1 = jnp.dot(x1_ref[...], w1_ref[...], preferred_element_type=jnp.float32)
    m2 = jnp.dot(x2_ref[...], w2_ref[...], preferred_element_type=jnp.float32)
    a = a_ref[...]
    s1 = jnp.sum(a * jnp.tanh(m1), axis=-1)
    s2 = jnp.sum(a * jnp.tanh(m2), axis=-1)
    l1 = jnp.where(s1 >= 0, s1, 0.2 * s1)
    l2 = jnp.where(s2 >= 0, s2, 0.2 * s2)
    w = jax.nn.sigmoid(l1 - l2)[:, None]
    h = w * m1 + (1.0 - w) * m2
    hP_ref[...] = h
    hPs_ref[0] = h[:, :32]
    hPs_ref[1] = h[:, 32:]
    mA = jnp.dot(x3_ref[...], w3_ref[...], preferred_element_type=jnp.float32)
    hA_ref[...] = mA
    hAs_ref[0] = mA[:, :32]
    hAs_ref[1] = mA[:, 32:]


def _fuse_tc(x1, x2, x3, w1, w2, w3, a):
    grid = (N_P // _RB,)
    return pl.pallas_call(
        _fuse_body,
        grid=grid,
        in_specs=[
            pl.BlockSpec((_RB, 256), lambda i: (i, 0)),
            pl.BlockSpec((_RB, 384), lambda i: (i, 0)),
            pl.BlockSpec((_RB, 256), lambda i: (i, 0)),
            pl.BlockSpec((256, D), lambda i: (0, 0)),
            pl.BlockSpec((384, D), lambda i: (0, 0)),
            pl.BlockSpec((256, D), lambda i: (0, 0)),
            pl.BlockSpec((1, D), lambda i: (0, 0)),
        ],
        out_specs=[
            pl.BlockSpec((_RB, D), lambda i: (i, 0)),
            pl.BlockSpec((NC, _RB, 32), lambda i: (0, i, 0)),
            pl.BlockSpec((_RB, D), lambda i: (i, 0)),
            pl.BlockSpec((NC, _RB, 32), lambda i: (0, i, 0)),
        ],
        out_shape=[
            jax.ShapeDtypeStruct((N_P, D), jnp.float32),
            jax.ShapeDtypeStruct((NC, N_P, 32), jnp.float32),
            jax.ShapeDtypeStruct((N_A, D), jnp.float32),
            jax.ShapeDtypeStruct((NC, N_A, 32), jnp.float32),
        ],
    )(x1, x2, x3, w1, w2, w3, a)


_NB = 1024


def _norm_split_body(s_ref, deg_ref, o_ref):
    d = jnp.maximum(deg_ref[...][:, :1], 1.0)
    o_ref[0] = s_ref[0] / d
    o_ref[1] = s_ref[1] / d


def _norm_split_tc(sums, deg):
    return pl.pallas_call(
        _norm_split_body,
        grid=(NPAD // _NB,),
        in_specs=[
            pl.BlockSpec((NC, _NB, 32), lambda i: (0, i, 0)),
            pl.BlockSpec((_NB, 16), lambda i: (i, 0)),
        ],
        out_specs=pl.BlockSpec((NC, _NB, 32), lambda i: (0, i, 0)),
        out_shape=jax.ShapeDtypeStruct((NC, NPAD, 32), jnp.float32),
    )(sums, deg)


def _norm_final_body(sa_ref, sb_ref, deg_ref, oa_ref, ob_ref):
    d = jnp.maximum(deg_ref[...][:, :1], 1.0)
    oa_ref[...] = jnp.concatenate([sa_ref[0] / d, sa_ref[1] / d], axis=1)
    ob_ref[...] = jnp.concatenate([sb_ref[0] / d, sb_ref[1] / d], axis=1)


def _norm_final_tc(sums_a, sums_b, deg):
    return pl.pallas_call(
        _norm_final_body,
        grid=(NPAD // _NB,),
        in_specs=[
            pl.BlockSpec((NC, _NB, 32), lambda i: (0, i, 0)),
            pl.BlockSpec((NC, _NB, 32), lambda i: (0, i, 0)),
            pl.BlockSpec((_NB, 16), lambda i: (i, 0)),
        ],
        out_specs=[
            pl.BlockSpec((_NB, D), lambda i: (i, 0)),
            pl.BlockSpec((_NB, D), lambda i: (i, 0)),
        ],
        out_shape=[
            jax.ShapeDtypeStruct((NPAD, D), jnp.float32),
            jax.ShapeDtypeStruct((NPAD, D), jnp.float32),
        ],
    )(sums_a, sums_b, deg)


def _pad_idx(idx, fill):
    idx = idx.astype(jnp.int32)
    pad = jnp.full((EPAD - E,), fill, jnp.int32)
    return jnp.concatenate([idx, pad]).reshape(EROWS, 128)


def kernel(feat_P_PA, feat_P_PAP, feat_A_AP, src_AP, dst_AP, src_PA, dst_PA,
           W_PA, W_PAP, W_AP, a_P, a_A):
    srcAP = _pad_idx(src_AP, 0)
    dstAP = _pad_idx(dst_AP, N_P)
    srcPA = _pad_idx(src_PA, 0)
    dstPA = _pad_idx(dst_PA, N_A)

    zeros32 = jnp.zeros((NPAD, 32), jnp.float32)
    zeros16 = jnp.zeros((NPAD, 16), jnp.float32)
    ones16 = jnp.ones((128, 16), jnp.float32)

    h_P, h_P_split, h_A, h_A_split = _fuse_tc(
        feat_P_PA, feat_P_PAP, feat_A_AP, W_PA, W_PAP, W_AP, a_P)

    deg_P, deg_A = _degrees_sc(dstAP, dstPA, ones16, zeros16)

    s_PA = _seg_sum_sc(h_A_split[0], h_A_split[1], srcAP, dstAP, zeros32)
    s_AP = _seg_sum_sc(h_P_split[0], h_P_split[1], srcPA, dstPA, zeros32)
    f_AP_split = _norm_split_tc(s_AP, deg_A)

    s_PAP = _seg_sum_sc(f_AP_split[0], f_AP_split[1], srcAP, dstAP, zeros32)

    f_PA, f_PAP = _norm_final_tc(s_PA, s_PAP, deg_P)

    return jnp.stack([h_P, f_PA[:N_P], f_PAP[:N_P]])
